# baseline (speedup 1.0000x reference)
import functools
import math

import jax
import jax.numpy as jnp
import numpy as np
from jax import lax
from jax.experimental import pallas as pl
from jax.experimental.pallas import tpu as pltpu
from jax.experimental.pallas import tpu_sc as plsc

_LANES = 16
_SUB = 40


@functools.lru_cache(maxsize=None)
def _make_kernel(total, S, D, scale):
    info = plsc.get_sparse_core_info()
    NC, NS = info.num_cores, info.num_subcores
    NW = NC * NS
    per_w = total // NW
    assert per_w * NW == total and per_w % S == 0
    chunks = per_w // S
    nsub = S // _SUB
    mesh = plsc.VectorSubcoreMesh(core_axis_name="c", subcore_axis_name="s")

    @functools.partial(
        pl.kernel,
        mesh=mesh,
        compiler_params=pltpu.CompilerParams(use_tc_tiling_on_sc=False),
        out_type=jax.ShapeDtypeStruct((total, D), jnp.float32),
        scratch_types=[
            pltpu.VMEM((S,), jnp.int32),
            pltpu.VMEM((S, D), jnp.float32),
            pltpu.VMEM((S, D), jnp.float32),
            pltpu.SemaphoreType.DMA,
        ],
    )
    def k(x_hbm, table_hbm, pos_hbm, out_hbm, idx_v, rows_v, pos_v, sem):
        wid = lax.axis_index("s") * NC + lax.axis_index("c")
        base = wid * per_w
        pltpu.sync_copy(pos_hbm, pos_v)

        def chunk_body(i, carry):
            off = base + i * S
            pltpu.sync_copy(x_hbm.at[pl.ds(off, S)], idx_v)
            copies = []
            for j in range(nsub):
                sl = pl.ds(j * _SUB, _SUB)
                copies.append(
                    pltpu.async_copy(
                        table_hbm.at[idx_v.at[sl]], rows_v.at[sl], sem))
            for c in copies:
                c.wait()

            def row_body(r, c2):
                for c in range(D // _LANES):
                    csl = pl.ds(c * _LANES, _LANES)
                    rows_v[r, csl] = rows_v[r, csl] * scale + pos_v[r, csl]
                return c2

            lax.fori_loop(0, S, row_body, 0)
            pltpu.sync_copy(rows_v, out_hbm.at[pl.ds(off, S)])
            return carry

        lax.fori_loop(0, chunks, chunk_body, 0)

    return k


def kernel(x, table, pos_encoding, training=False):
    B, S = x.shape
    D = table.shape[1]
    scale = float(np.float32(math.sqrt(D)))
    pos = pos_encoding[0, :S, :].astype(jnp.float32)
    k = _make_kernel(B * S, S, D, scale)
    out = k(x.reshape(-1), table, pos)
    return out.reshape(B, S, D)


# speedup vs baseline: 1.2144x; 1.2144x over previous
import functools
import math

import jax
import jax.numpy as jnp
import numpy as np
from jax import lax
from jax.experimental import pallas as pl
from jax.experimental.pallas import tpu as pltpu
from jax.experimental.pallas import tpu_sc as plsc

_LANES = 16
_SUB = 40
_NBUF = 4


@functools.lru_cache(maxsize=None)
def _make_kernel(total, S, D, scale):
    info = plsc.get_sparse_core_info()
    NC, NS = info.num_cores, info.num_subcores
    NW = NC * NS
    per_w = total // NW
    assert per_w * NW == total and per_w % S == 0
    chunks = per_w // S
    assert chunks % _NBUF == 0 and chunks >= 2 * _NBUF
    nsub = S // _SUB
    mesh = plsc.VectorSubcoreMesh(core_axis_name="c", subcore_axis_name="s")

    @functools.partial(
        pl.kernel,
        mesh=mesh,
        compiler_params=pltpu.CompilerParams(use_tc_tiling_on_sc=False),
        out_type=jax.ShapeDtypeStruct((total, D), jnp.float32),
        scratch_types=[
            pltpu.VMEM((per_w,), jnp.int32),
            pltpu.VMEM((_NBUF, S, D), jnp.float32),
            pltpu.VMEM((S, D), jnp.float32),
        ]
        + [pltpu.SemaphoreType.DMA] * (2 * _NBUF),
    )
    def k(x_hbm, table_hbm, pos_hbm, out_hbm, idx_v, rows, pos_v, *sems):
        gsem = sems[:_NBUF]
        ssem = sems[_NBUF:]
        wid = lax.axis_index("s") * NC + lax.axis_index("c")
        base = wid * per_w
        pltpu.sync_copy(x_hbm.at[pl.ds(base, per_w)], idx_v)
        pltpu.sync_copy(pos_hbm, pos_v)

        def fire_gather(c, b):
            off = pl.multiple_of(c * S, S)
            for j in range(nsub):
                pltpu.async_copy(
                    table_hbm.at[idx_v.at[pl.ds(off + j * _SUB, _SUB)]],
                    rows.at[b].at[pl.ds(j * _SUB, _SUB)],
                    gsem[b])

        def drain_gather(b):
            pltpu.make_async_copy(
                table_hbm.at[pl.ds(0, S)], rows.at[b], gsem[b]).wait()

        def drain_store(b):
            pltpu.make_async_copy(
                rows.at[b], out_hbm.at[pl.ds(0, S)], ssem[b]).wait()

        def compute(b):
            def qbody(q, carry):
                r0 = q * 4
                for dr in range(4):
                    r = r0 + dr
                    for col in range(D // _LANES):
                        sl = pl.ds(col * _LANES, _LANES)
                        rows[b, r, sl] = rows[b, r, sl] * scale + pos_v[r, sl]
                return carry
            lax.fori_loop(0, S // 4, qbody, 0)

        fire_gather(0, 0)
        fire_gather(1, 1)

        def loop_body(p, carry):
            for b in range(_NBUF):
                c = p * _NBUF + b
                drain_gather(b)
                compute(b)
                pltpu.async_copy(
                    rows.at[b],
                    out_hbm.at[pl.ds(pl.multiple_of(base + c * S, S), S)],
                    ssem[b])
                c2 = c + 2
                b2 = (b + 2) % _NBUF

                @pl.when(c2 < chunks)
                def _():
                    @pl.when(c >= 2)
                    def _():
                        drain_store(b2)
                    fire_gather(c2, b2)
            return carry

        lax.fori_loop(0, chunks // _NBUF, loop_body, 0)
        for b in range(_NBUF):
            drain_store(b)

    return k


def kernel(x, table, pos_encoding, training=False):
    B, S = x.shape
    D = table.shape[1]
    scale = float(np.float32(math.sqrt(D)))
    pos = pos_encoding[0, :S, :].astype(jnp.float32)
    k = _make_kernel(B * S, S, D, scale)
    out = k(x.reshape(-1), table, pos)
    return out.reshape(B, S, D)
